# baseline (speedup 1.0000x reference)
import functools

import jax
import jax.numpy as jnp
import numpy as np
from jax import lax
from jax.experimental import pallas as pl
from jax.experimental.pallas import tpu as pltpu
from jax.experimental.pallas import tpu_sc as plsc

N_TOTAL = 20000
N_SUB = 10000
E = 320000
D_FEAT = 128
HID = 128
NL = 4
DK = 32
OUT_DIM = 64
T = 2048

NC = 2
NS = 16
CH = 32

NPAD = 10240
PADROW = 10200
NCHUNK = -2 * (-(E // NS) // (2 * CH))
EPT = NCHUNK * CH
E_PAD = EPT * NS
RPT = NPAD // NS
DR = NPAD // 32
TPT = T // NS

_mesh = plsc.VectorSubcoreMesh(core_axis_name="c", subcore_axis_name="s")
_sc_params = pltpu.CompilerParams(needs_layout_passes=False)


def _proj_body(x_ref, w_ref, b_ref, o_ref):
    o_ref[...] = jnp.dot(x_ref[0], w_ref[0],
                         preferred_element_type=jnp.float32) + b_ref[0]


def _type_proj(feats01, W, b):
    return pl.pallas_call(
        _proj_body,
        grid=(2, 5),
        in_specs=[
            pl.BlockSpec((1, 2000, D_FEAT), lambda t, i: (t, i, 0)),
            pl.BlockSpec((1, D_FEAT, HID), lambda t, i: (t, 0, 0)),
            pl.BlockSpec((1, 1, HID), lambda t, i: (t, 0, 0)),
        ],
        out_specs=pl.BlockSpec((2000, HID), lambda t, i: (t * 5 + i, 0)),
        out_shape=jax.ShapeDtypeStruct((N_TOTAL, HID), jnp.float32),
    )(feats01, W, b)


def _latent_body(x_ref, w_ref, b_ref, o_ref):
    o_ref[0] = jnp.tanh(jnp.dot(x_ref[0], w_ref[0],
                                preferred_element_type=jnp.float32)
                        + b_ref[0])


def _latent_proj(feats, Wf, bf):
    return pl.pallas_call(
        _latent_body,
        grid=(2, 8),
        in_specs=[
            pl.BlockSpec((1, NPAD // 8, HID), lambda b, i: (b, i, 0)),
            pl.BlockSpec((1, HID, HID), lambda b, i: (b, 0, 0)),
            pl.BlockSpec((1, 1, HID), lambda b, i: (b, 0, 0)),
        ],
        out_specs=pl.BlockSpec((1, NPAD // 8, HID), lambda b, i: (b, i, 0)),
        out_shape=jax.ShapeDtypeStruct((2, NPAD, HID), jnp.float32),
    )(feats, Wf, bf)


def _final_body(n_ref, d_ref, r_ref, w_ref, b_ref, o_ref):
    denr = jnp.dot(d_ref[0], r_ref[...], preferred_element_type=jnp.float32)
    emb = n_ref[0] / (denr + 1e-9)
    o_ref[0] = jnp.dot(emb, w_ref[0],
                       preferred_element_type=jnp.float32) + b_ref[0]


def _final_proj(num_t, den_t, rep, W, b):
    return pl.pallas_call(
        _final_body,
        grid=(2,),
        in_specs=[
            pl.BlockSpec((1, T, HID), lambda b_: (b_, 0, 0)),
            pl.BlockSpec((1, T, NL), lambda b_: (b_, 0, 0)),
            pl.BlockSpec((NL, HID), lambda b_: (0, 0)),
            pl.BlockSpec((1, HID, OUT_DIM), lambda b_: (b_, 0, 0)),
            pl.BlockSpec((1, 1, OUT_DIM), lambda b_: (b_, 0, 0)),
        ],
        out_specs=pl.BlockSpec((1, T, OUT_DIM), lambda b_: (b_, 0, 0)),
        out_shape=jax.ShapeDtypeStruct((2, T, OUT_DIM), jnp.float32),
    )(num_t, den_t, rep, W, b)


@functools.partial(
    pl.kernel, mesh=_mesh, compiler_params=_sc_params,
    out_type=pltpu.HBM((2 * NPAD, HID), jnp.float32),
    scratch_types=[
        pltpu.VMEM((128,), jnp.int32),
        pltpu.VMEM((128, HID), jnp.float32),
        pltpu.SemaphoreType.DMA,
    ],
)
def _node_gather(trans_hbm, idx_hbm, out_hbm, idx_v, rows_v, sem):
    cid = lax.axis_index("c")
    sid = lax.axis_index("s")
    base = (cid * NS + sid) * (2 * NPAD // (NC * NS))

    def chunk(i, carry):
        off = base + i * 128
        pltpu.sync_copy(idx_hbm.at[pl.ds(off, 128)], idx_v)
        pltpu.async_copy(trans_hbm.at[idx_v], rows_v, sem).wait()
        pltpu.sync_copy(rows_v, out_hbm.at[pl.ds(off, 128)])
        return carry

    lax.fori_loop(0, 2 * NPAD // (NC * NS) // 128, chunk, 0)


@functools.partial(
    pl.kernel, mesh=_mesh, compiler_params=_sc_params,
    out_type=[
        pltpu.HBM((2 * T, HID), jnp.float32),
        pltpu.HBM((2, T * NL), jnp.float32),
    ],
    scratch_types=[
        pltpu.VMEM_SHARED((NPAD, HID), jnp.float32),
        pltpu.VMEM_SHARED((DR, HID), jnp.float32),
        pltpu.VMEM((CH,), jnp.int32),
        pltpu.VMEM((CH,), jnp.int32),
        pltpu.VMEM((CH,), jnp.int32),
        pltpu.VMEM((CH,), jnp.int32),
        pltpu.VMEM((CH, HID), jnp.float32),
        pltpu.VMEM((CH, HID), jnp.float32),
        pltpu.VMEM((CH, HID), jnp.float32),
        pltpu.VMEM((CH, HID), jnp.float32),
        pltpu.VMEM((CH,), jnp.int32),
        pltpu.VMEM((CH,), jnp.int32),
        pltpu.VMEM((CH,), jnp.int32),
        pltpu.VMEM((CH,), jnp.int32),
        pltpu.VMEM((CH, HID), jnp.float32),
        pltpu.VMEM((CH, HID), jnp.float32),
        pltpu.VMEM((CH, HID), jnp.float32),
        pltpu.VMEM((CH, HID), jnp.float32),
        pltpu.VMEM((TPT,), jnp.int32),
        pltpu.VMEM((TPT * NL,), jnp.float32),
        pltpu.SemaphoreType.DMA,
        pltpu.SemaphoreType.DMA,
        pltpu.SemaphoreType.DMA,
        pltpu.SemaphoreType.DMA,
        pltpu.SemaphoreType.DMA,
        pltpu.SemaphoreType.DMA,
    ],
)
def _edge_pass(z_hbm, src_hbm, dstz_hbm, targ_hbm, zeros_hbm,
               onum_hbm, oden_hbm, num_sh, den_sh,
               s0_v, dz0_v, s1_v, dz1_v,
               zs0_v, zd0_v, zs1_v, zd1_v,
               dlc0_v, ddc0_v, dlc1_v, ddc1_v,
               outv0, outd0, outv1, outd1, tg_v, dgath_v,
               isem0, isem1, zsem0, zsem1, ssem0, ssem1):
    cid = lax.axis_index("c")
    sid = lax.axis_index("s")

    pltpu.sync_copy(zeros_hbm.at[pl.ds(sid * RPT, RPT)],
                    num_sh.at[pl.ds(sid * RPT, RPT)])

    @pl.when(sid < DR // 32)
    def _zero_den():
        pltpu.sync_copy(zeros_hbm.at[pl.ds(sid * 32, 32)],
                        den_sh.at[pl.ds(sid * 32, 32)])

    plsc.subcore_barrier()

    base_e = cid * E_PAD + sid * EPT
    iota = lax.iota(jnp.int32, 16)
    zero16 = jnp.zeros((16,), jnp.float32)
    perms = [(iota ^ jnp.full((16,), s, jnp.int32)).reshape(16, 1)
             for s in (1, 2, 4, 8)]
    _dn = lax.GatherDimensionNumbers(
        offset_dims=(), collapsed_slice_dims=(0,), start_index_map=(0,))

    def gat(v, p):
        return lax.gather(v, p, _dn, slice_sizes=(1,),
                          mode=lax.GatherScatterMode.PROMISE_IN_BOUNDS)

    def lane_sum(v):
        for p in perms:
            v = v + gat(v, p)
        return v

    bufs = ((s0_v, dz0_v, None, isem0, zs0_v, zd0_v, zsem0,
             outv0, outd0, dlc0_v, ddc0_v, ssem0),
            (s1_v, dz1_v, None, isem1, zs1_v, zd1_v, zsem1,
             outv1, outd1, dlc1_v, ddc1_v, ssem1))
    zoff = cid * NPAD

    def issue_idx(c, b):
        eb = base_e + c * CH
        pltpu.async_copy(src_hbm.at[pl.ds(eb, CH)], b[0], b[3])
        pltpu.async_copy(dstz_hbm.at[pl.ds(eb, CH)], b[1], b[3])

    def wait_idx(b):
        for r in (b[0], b[1]):
            pltpu.make_async_copy(src_hbm.at[pl.ds(0, CH)], r, b[3]).wait()

    def issue_z(b):
        pltpu.async_copy(z_hbm.at[b[0]], b[4], b[6])
        pltpu.async_copy(z_hbm.at[b[1]], b[5], b[6])

    def wait_z(b):
        for r in (b[4], b[5]):
            pltpu.make_async_copy(z_hbm.at[pl.ds(0, CH)], r, b[6]).wait()

    def prep(b):
        didxz_v, dlc_v, ddc_v = b[1], b[9], b[10]

        def body(t, c2):
            d16 = didxz_v[pl.ds(16 * t, 16)] - zoff
            dlc_v[pl.ds(16 * t, 16)] = d16
            ddc_v[pl.ds(16 * t, 16)] = lax.shift_right_logical(d16, 5)
            return c2

        lax.fori_loop(0, CH // 16, body, 0)

    def wait_scatter(b):
        pltpu.make_async_copy(outv0, num_sh.at[b[9]], b[11]).wait()
        pltpu.make_async_copy(outd0, den_sh.at[b[10]], b[11]).wait()

    def edges_and_scatter(b):
        zsrc_v, zdst_v = b[4], b[5]
        outv, outd_v, dlc_v = b[7], b[8], b[9]

        @plsc.parallel_loop(0, CH, 1, unroll=2)
        def edge(e):
            zs = [zsrc_v[e, pl.ds(16 * j, 16)] for j in range(8)]
            zd = [zdst_v[e, pl.ds(16 * j, 16)] for j in range(8)]
            dsts16 = dlc_v[pl.ds((e >> 4) * 16, 16)]
            dstv = gat(dsts16, jnp.full((16, 1), e & 15, jnp.int32))
            ofs = (dstv & 3) * 4
            slotv = lax.shift_right_logical(dstv & 31, 2)
            wden = zero16
            for k in range(NL):
                p = zs[2 * k] * zd[2 * k] + zs[2 * k + 1] * zd[2 * k + 1]
                dv = lane_sum(p)
                wv = jnp.exp(jnp.where(dv > 0.0, dv, 0.2 * dv))
                outv[e, pl.ds(32 * k, 16)] = wv * zs[2 * k]
                outv[e, pl.ds(32 * k + 16, 16)] = wv * zs[2 * k + 1]
                wden = jnp.where(iota == ofs + k, wv, wden)
            for j in range(8):
                outd_v[e, pl.ds(16 * j, 16)] = jnp.where(
                    slotv == j, wden, zero16)

        pltpu.async_copy(outv, num_sh.at[dlc_v], b[11], add=True)
        pltpu.async_copy(outd_v, den_sh.at[b[10]], b[11], add=True)

    issue_idx(0, bufs[0])
    wait_idx(bufs[0])
    issue_z(bufs[0])
    issue_idx(1, bufs[1])
    for b in bufs:
        pltpu.async_copy(zeros_hbm.at[pl.ds(0, CH)], b[7], b[11])
        pltpu.async_copy(zeros_hbm.at[pl.ds(0, CH)], b[8], b[11])

    def phase(t, b, bo):
        wait_idx(bo)
        issue_z(bo)
        wait_z(b)
        wait_scatter(b)
        prep(b)
        issue_idx(jnp.minimum(t + 2, NCHUNK - 1), b)
        edges_and_scatter(b)

    def piter(g, carry):
        t = 2 * g
        phase(t, bufs[0], bufs[1])
        phase(t + 1, bufs[1], bufs[0])
        return carry

    lax.fori_loop(0, NCHUNK // 2, piter, 0)
    wait_idx(bufs[1])
    wait_z(bufs[0])
    wait_scatter(bufs[0])
    wait_scatter(bufs[1])
    plsc.subcore_barrier()

    tb = cid * T + sid * TPT
    pltpu.sync_copy(targ_hbm.at[pl.ds(tb, TPT)], tg_v)
    for hp in range(TPT // 32):
        pltpu.async_copy(num_sh.at[tg_v.at[pl.ds(hp * 32, 32)]],
                         zs0_v.at[pl.ds(0, 32)], zsem0).wait()
        pltpu.sync_copy(zs0_v.at[pl.ds(0, 32)],
                        onum_hbm.at[pl.ds(tb + hp * 32, 32)])

        def trow(t, c2):
            t16 = tg_v[pl.ds(hp * 32 + 16 * t, 16)]
            ddc0_v[pl.ds(16 * t, 16)] = lax.shift_right_logical(t16, 5)
            return c2

        lax.fori_loop(0, 2, trow, 0)
        pltpu.async_copy(den_sh.at[ddc0_v.at[pl.ds(0, 32)]],
                         zd0_v.at[pl.ds(0, 32)], zsem0).wait()

        def dext(m, c2):
            t16 = tg_v[pl.ds(hp * 32 + (m // 4) * 16, 16)]
            sel = (iota // 4 + (4 * m) % 16).reshape(16, 1)
            tsel = gat(t16, sel)
            rowv = iota // 4 + 4 * m
            colv = (tsel & 31) * 4 + (iota & 3)
            vals = plsc.load_gather(zd0_v, [rowv, colv])
            dgath_v[pl.ds((hp * 8 + m) * 16, 16)] = vals
            return c2

        lax.fori_loop(0, 8, dext, 0)
    pltpu.sync_copy(dgath_v, oden_hbm.at[cid, pl.ds(sid * TPT * NL, TPT * NL)])


def kernel(feat0, feat1, type_mask, node_idx_gene, node_idx_dis,
           edge_index_gene, edge_index_dis, target_idx_gene, target_idx_dis,
           fc_type_W, fc_type_b, gene_Wf, gene_bf, gene_fc1_W, gene_fc1_b,
           gene_fc2_W, gene_fcout_W, gene_fcout_b, dis_Wf, dis_bf, dis_fc1_W,
           dis_fc1_b, dis_fc2_W, dis_fcout_W, dis_fcout_b):
    f32 = jnp.float32
    i32 = jnp.int32

    feats01 = jnp.stack([feat0, feat1])
    trans = _type_proj(feats01, fc_type_W, fc_type_b.reshape(2, 1, HID))

    zpad = jnp.zeros((NPAD - N_SUB,), i32)
    node_cat = jnp.concatenate(
        [node_idx_gene, zpad, node_idx_dis, zpad]).astype(i32)
    feats = _node_gather(trans, node_cat)

    Wf_flat = jnp.stack([
        jnp.concatenate([gene_Wf[k] for k in range(NL)], axis=1),
        jnp.concatenate([dis_Wf[k] for k in range(NL)], axis=1),
    ])
    bf_flat = jnp.stack([gene_bf.reshape(1, HID), dis_bf.reshape(1, HID)])
    Z = _latent_proj(feats.reshape(2, NPAD, HID), Wf_flat, bf_flat)
    Zflat = Z.reshape(2 * NPAD, HID)

    npad_e = E_PAD - E

    def edge_arrays(ei, zoff):
        pad_g = jnp.full((npad_e,), zoff + PADROW, i32)
        src = jnp.concatenate([ei[0] + zoff, pad_g])
        dstz = jnp.concatenate([ei[1] + zoff, pad_g])
        return src, dstz

    sg, dzg = edge_arrays(edge_index_gene, 0)
    sd, dzd = edge_arrays(edge_index_dis, NPAD)
    src_cat = jnp.concatenate([sg, sd])
    dstz_cat = jnp.concatenate([dzg, dzd])
    targ_cat = jnp.concatenate([target_idx_gene, target_idx_dis]).astype(i32)
    zeros_init = jnp.zeros((NPAD, HID), f32)

    onum, oden = _edge_pass(Zflat, src_cat, dstz_cat, targ_cat, zeros_init)

    num_t = onum.reshape(2, T, HID)
    den_t = oden.reshape(2, T, NL)
    rep = np.zeros((NL, HID), np.float32)
    for k in range(NL):
        rep[k, k * DK:(k + 1) * DK] = 1.0
    foW = jnp.stack([gene_fcout_W, dis_fcout_W])
    fob = jnp.stack([gene_fcout_b.reshape(1, OUT_DIM),
                     dis_fcout_b.reshape(1, OUT_DIM)])
    res = _final_proj(num_t, den_t, jnp.asarray(rep), foW, fob)
    return res[0], res[1]


# speedup vs baseline: 1.0376x; 1.0376x over previous
import functools

import jax
import jax.numpy as jnp
import numpy as np
from jax import lax
from jax.experimental import pallas as pl
from jax.experimental.pallas import tpu as pltpu
from jax.experimental.pallas import tpu_sc as plsc

N_TOTAL = 20000
N_SUB = 10000
E = 320000
D_FEAT = 128
HID = 128
NL = 4
DK = 32
OUT_DIM = 64
T = 2048

NC = 2
NS = 16
CH = 32

NPAD = 10240
PADROW = 10200
NCHUNK = -2 * (-(E // NS) // (2 * CH))
EPT = NCHUNK * CH
E_PAD = EPT * NS
RPT = NPAD // NS
DR = NPAD // 32
TPT = T // NS

_mesh = plsc.VectorSubcoreMesh(core_axis_name="c", subcore_axis_name="s")
_sc_params = pltpu.CompilerParams(needs_layout_passes=False)


def _proj_body(x_ref, w_ref, b_ref, o_ref):
    o_ref[...] = jnp.dot(x_ref[0], w_ref[0],
                         preferred_element_type=jnp.float32) + b_ref[0]


def _type_proj(feats01, W, b):
    return pl.pallas_call(
        _proj_body,
        grid=(2, 5),
        in_specs=[
            pl.BlockSpec((1, 2000, D_FEAT), lambda t, i: (t, i, 0)),
            pl.BlockSpec((1, D_FEAT, HID), lambda t, i: (t, 0, 0)),
            pl.BlockSpec((1, 1, HID), lambda t, i: (t, 0, 0)),
        ],
        out_specs=pl.BlockSpec((2000, HID), lambda t, i: (t * 5 + i, 0)),
        out_shape=jax.ShapeDtypeStruct((N_TOTAL, HID), jnp.float32),
    )(feats01, W, b)


def _latent_body(x_ref, w_ref, b_ref, o_ref):
    o_ref[0] = jnp.tanh(jnp.dot(x_ref[0], w_ref[0],
                                preferred_element_type=jnp.float32)
                        + b_ref[0])


def _latent_proj(feats, Wf, bf):
    return pl.pallas_call(
        _latent_body,
        grid=(2, 8),
        in_specs=[
            pl.BlockSpec((1, NPAD // 8, HID), lambda b, i: (b, i, 0)),
            pl.BlockSpec((1, HID, HID), lambda b, i: (b, 0, 0)),
            pl.BlockSpec((1, 1, HID), lambda b, i: (b, 0, 0)),
        ],
        out_specs=pl.BlockSpec((1, NPAD // 8, HID), lambda b, i: (b, i, 0)),
        out_shape=jax.ShapeDtypeStruct((2, NPAD, HID), jnp.float32),
    )(feats, Wf, bf)


def _final_body(n_ref, d_ref, r_ref, w_ref, b_ref, o_ref):
    denr = jnp.dot(d_ref[0], r_ref[...], preferred_element_type=jnp.float32)
    emb = n_ref[0] / (denr + 1e-9)
    o_ref[0] = jnp.dot(emb, w_ref[0],
                       preferred_element_type=jnp.float32) + b_ref[0]


def _final_proj(num_t, den_t, rep, W, b):
    return pl.pallas_call(
        _final_body,
        grid=(2,),
        in_specs=[
            pl.BlockSpec((1, T, HID), lambda b_: (b_, 0, 0)),
            pl.BlockSpec((1, T, NL), lambda b_: (b_, 0, 0)),
            pl.BlockSpec((NL, HID), lambda b_: (0, 0)),
            pl.BlockSpec((1, HID, OUT_DIM), lambda b_: (b_, 0, 0)),
            pl.BlockSpec((1, 1, OUT_DIM), lambda b_: (b_, 0, 0)),
        ],
        out_specs=pl.BlockSpec((1, T, OUT_DIM), lambda b_: (b_, 0, 0)),
        out_shape=jax.ShapeDtypeStruct((2, T, OUT_DIM), jnp.float32),
    )(num_t, den_t, rep, W, b)


@functools.partial(
    pl.kernel, mesh=_mesh, compiler_params=_sc_params,
    out_type=pltpu.HBM((2 * NPAD, HID), jnp.float32),
    scratch_types=[
        pltpu.VMEM((128,), jnp.int32),
        pltpu.VMEM((128, HID), jnp.float32),
        pltpu.SemaphoreType.DMA,
    ],
)
def _node_gather(trans_hbm, idx_hbm, out_hbm, idx_v, rows_v, sem):
    cid = lax.axis_index("c")
    sid = lax.axis_index("s")
    base = (cid * NS + sid) * (2 * NPAD // (NC * NS))

    def chunk(i, carry):
        off = base + i * 128
        pltpu.sync_copy(idx_hbm.at[pl.ds(off, 128)], idx_v)
        pltpu.async_copy(trans_hbm.at[idx_v], rows_v, sem).wait()
        pltpu.sync_copy(rows_v, out_hbm.at[pl.ds(off, 128)])
        return carry

    lax.fori_loop(0, 2 * NPAD // (NC * NS) // 128, chunk, 0)


@functools.partial(
    pl.kernel, mesh=_mesh, compiler_params=_sc_params,
    out_type=[
        pltpu.HBM((2 * T, HID), jnp.float32),
        pltpu.HBM((2, T * NL), jnp.float32),
    ],
    scratch_types=[
        pltpu.VMEM_SHARED((NPAD, HID), jnp.float32),
        pltpu.VMEM_SHARED((DR, HID), jnp.float32),
        pltpu.VMEM((CH,), jnp.int32),
        pltpu.VMEM((CH,), jnp.int32),
        pltpu.VMEM((CH,), jnp.int32),
        pltpu.VMEM((CH,), jnp.int32),
        pltpu.VMEM((CH, HID), jnp.float32),
        pltpu.VMEM((CH, HID), jnp.float32),
        pltpu.VMEM((CH, HID), jnp.float32),
        pltpu.VMEM((CH, HID), jnp.float32),
        pltpu.VMEM((CH,), jnp.int32),
        pltpu.VMEM((CH,), jnp.int32),
        pltpu.VMEM((CH,), jnp.int32),
        pltpu.VMEM((CH,), jnp.int32),
        pltpu.VMEM((CH, HID), jnp.float32),
        pltpu.VMEM((CH, HID), jnp.float32),
        pltpu.VMEM((CH, HID), jnp.float32),
        pltpu.VMEM((CH, HID), jnp.float32),
        pltpu.VMEM((TPT,), jnp.int32),
        pltpu.VMEM((TPT * NL,), jnp.float32),
        pltpu.SemaphoreType.DMA,
        pltpu.SemaphoreType.DMA,
        pltpu.SemaphoreType.DMA,
        pltpu.SemaphoreType.DMA,
        pltpu.SemaphoreType.DMA,
        pltpu.SemaphoreType.DMA,
    ],
)
def _edge_pass(z_hbm, src_hbm, dstz_hbm, targ_hbm, zeros_hbm,
               onum_hbm, oden_hbm, num_sh, den_sh,
               s0_v, dz0_v, s1_v, dz1_v,
               zs0_v, zd0_v, zs1_v, zd1_v,
               dlc0_v, ddc0_v, dlc1_v, ddc1_v,
               outv0, outd0, outv1, outd1, tg_v, dgath_v,
               isem0, isem1, zsem0, zsem1, ssem0, ssem1):
    cid = lax.axis_index("c")
    sid = lax.axis_index("s")

    pltpu.sync_copy(zeros_hbm.at[pl.ds(sid * RPT, RPT)],
                    num_sh.at[pl.ds(sid * RPT, RPT)])

    @pl.when(sid < DR // 32)
    def _zero_den():
        pltpu.sync_copy(zeros_hbm.at[pl.ds(sid * 32, 32)],
                        den_sh.at[pl.ds(sid * 32, 32)])

    plsc.subcore_barrier()

    base_e = cid * E_PAD + sid * EPT
    iota = lax.iota(jnp.int32, 16)
    zero16 = jnp.zeros((16,), jnp.float32)
    perms = [(iota ^ jnp.full((16,), s, jnp.int32)).reshape(16, 1)
             for s in (1, 2, 4, 8)]
    _dn = lax.GatherDimensionNumbers(
        offset_dims=(), collapsed_slice_dims=(0,), start_index_map=(0,))

    def gat(v, p):
        return lax.gather(v, p, _dn, slice_sizes=(1,),
                          mode=lax.GatherScatterMode.PROMISE_IN_BOUNDS)

    def lane_sum(v):
        for p in perms:
            v = v + gat(v, p)
        return v

    bufs = ((s0_v, dz0_v, None, isem0, zs0_v, zd0_v, zsem0,
             outv0, outd0, dlc0_v, ddc0_v, ssem0),
            (s1_v, dz1_v, None, isem1, zs1_v, zd1_v, zsem1,
             outv1, outd1, dlc1_v, ddc1_v, ssem1))
    zoff = cid * NPAD

    def issue_idx(c, b):
        eb = base_e + c * CH
        pltpu.async_copy(src_hbm.at[pl.ds(eb, CH)], b[0], b[3])
        pltpu.async_copy(dstz_hbm.at[pl.ds(eb, CH)], b[1], b[3])

    def wait_idx(b):
        for r in (b[0], b[1]):
            pltpu.make_async_copy(src_hbm.at[pl.ds(0, CH)], r, b[3]).wait()

    def issue_z(b):
        pltpu.async_copy(z_hbm.at[b[0]], b[4], b[6])
        pltpu.async_copy(z_hbm.at[b[1]], b[5], b[6])

    def wait_z(b):
        for r in (b[4], b[5]):
            pltpu.make_async_copy(z_hbm.at[pl.ds(0, CH)], r, b[6]).wait()

    def prep(b):
        didxz_v, dlc_v, ddc_v = b[1], b[9], b[10]

        def body(t, c2):
            d16 = didxz_v[pl.ds(16 * t, 16)] - zoff
            dlc_v[pl.ds(16 * t, 16)] = d16
            ddc_v[pl.ds(16 * t, 16)] = lax.shift_right_logical(d16, 5)
            return c2

        lax.fori_loop(0, CH // 16, body, 0)

    def wait_scatter(b):
        pltpu.make_async_copy(outv0, num_sh.at[b[9]], b[11]).wait()
        pltpu.make_async_copy(outd0, den_sh.at[b[10]], b[11]).wait()

    def edges_and_scatter(b):
        zsrc_v, zdst_v = b[4], b[5]
        outv, outd_v, dlc_v = b[7], b[8], b[9]

        idxq = ((iota & 3) * 4).reshape(16, 1)
        bcast = [jnp.full((16, 1), 4 * k, jnp.int32) for k in range(NL)]

        @plsc.parallel_loop(0, CH, 1, unroll=2)
        def edge(e):
            zs = [zsrc_v[e, pl.ds(16 * j, 16)] for j in range(8)]
            zd = [zdst_v[e, pl.ds(16 * j, 16)] for j in range(8)]
            dsts16 = dlc_v[pl.ds((e >> 4) * 16, 16)]
            dstv = gat(dsts16, jnp.full((16, 1), e & 15, jnp.int32))
            ofs = (dstv & 3) * 4
            slotv = lax.shift_right_logical(dstv & 31, 2)
            ps = []
            for k in range(NL):
                p = zs[2 * k] * zd[2 * k] + zs[2 * k + 1] * zd[2 * k + 1]
                p = p + gat(p, perms[0])
                p = p + gat(p, perms[1])
                ps.append(p)
            q = jnp.where(iota < 4, gat(ps[0], idxq),
                          jnp.where(iota < 8, gat(ps[1], idxq),
                                    jnp.where(iota < 12, gat(ps[2], idxq),
                                              gat(ps[3], idxq))))
            q = q + gat(q, perms[0])
            q = q + gat(q, perms[1])
            wall = jnp.exp(jnp.where(q > 0.0, q, 0.2 * q))
            for k in range(NL):
                wv = gat(wall, bcast[k])
                outv[e, pl.ds(32 * k, 16)] = wv * zs[2 * k]
                outv[e, pl.ds(32 * k + 16, 16)] = wv * zs[2 * k + 1]
            wmask = (iota >= ofs) & (iota < ofs + 4)
            wsrc = (((iota - ofs) & 3) * 4).reshape(16, 1)
            wden = jnp.where(wmask, gat(wall, wsrc), zero16)
            for j in range(8):
                outd_v[e, pl.ds(16 * j, 16)] = jnp.where(
                    slotv == j, wden, zero16)

        pltpu.async_copy(outv, num_sh.at[dlc_v], b[11], add=True)
        pltpu.async_copy(outd_v, den_sh.at[b[10]], b[11], add=True)

    issue_idx(0, bufs[0])
    wait_idx(bufs[0])
    issue_z(bufs[0])
    issue_idx(1, bufs[1])
    for b in bufs:
        pltpu.async_copy(zeros_hbm.at[pl.ds(0, CH)], b[7], b[11])
        pltpu.async_copy(zeros_hbm.at[pl.ds(0, CH)], b[8], b[11])

    def phase(t, b, bo):
        wait_idx(bo)
        issue_z(bo)
        wait_z(b)
        wait_scatter(b)
        prep(b)
        issue_idx(jnp.minimum(t + 2, NCHUNK - 1), b)
        edges_and_scatter(b)

    def piter(g, carry):
        t = 2 * g
        phase(t, bufs[0], bufs[1])
        phase(t + 1, bufs[1], bufs[0])
        return carry

    lax.fori_loop(0, NCHUNK // 2, piter, 0)
    wait_idx(bufs[1])
    wait_z(bufs[0])
    wait_scatter(bufs[0])
    wait_scatter(bufs[1])
    plsc.subcore_barrier()

    tb = cid * T + sid * TPT
    pltpu.sync_copy(targ_hbm.at[pl.ds(tb, TPT)], tg_v)
    for hp in range(TPT // 32):
        pltpu.async_copy(num_sh.at[tg_v.at[pl.ds(hp * 32, 32)]],
                         zs0_v.at[pl.ds(0, 32)], zsem0).wait()
        pltpu.sync_copy(zs0_v.at[pl.ds(0, 32)],
                        onum_hbm.at[pl.ds(tb + hp * 32, 32)])

        def trow(t, c2):
            t16 = tg_v[pl.ds(hp * 32 + 16 * t, 16)]
            ddc0_v[pl.ds(16 * t, 16)] = lax.shift_right_logical(t16, 5)
            return c2

        lax.fori_loop(0, 2, trow, 0)
        pltpu.async_copy(den_sh.at[ddc0_v.at[pl.ds(0, 32)]],
                         zd0_v.at[pl.ds(0, 32)], zsem0).wait()

        def dext(m, c2):
            t16 = tg_v[pl.ds(hp * 32 + (m // 4) * 16, 16)]
            sel = (iota // 4 + (4 * m) % 16).reshape(16, 1)
            tsel = gat(t16, sel)
            rowv = iota // 4 + 4 * m
            colv = (tsel & 31) * 4 + (iota & 3)
            vals = plsc.load_gather(zd0_v, [rowv, colv])
            dgath_v[pl.ds((hp * 8 + m) * 16, 16)] = vals
            return c2

        lax.fori_loop(0, 8, dext, 0)
    pltpu.sync_copy(dgath_v, oden_hbm.at[cid, pl.ds(sid * TPT * NL, TPT * NL)])


def kernel(feat0, feat1, type_mask, node_idx_gene, node_idx_dis,
           edge_index_gene, edge_index_dis, target_idx_gene, target_idx_dis,
           fc_type_W, fc_type_b, gene_Wf, gene_bf, gene_fc1_W, gene_fc1_b,
           gene_fc2_W, gene_fcout_W, gene_fcout_b, dis_Wf, dis_bf, dis_fc1_W,
           dis_fc1_b, dis_fc2_W, dis_fcout_W, dis_fcout_b):
    f32 = jnp.float32
    i32 = jnp.int32

    feats01 = jnp.stack([feat0, feat1])
    trans = _type_proj(feats01, fc_type_W, fc_type_b.reshape(2, 1, HID))

    zpad = jnp.zeros((NPAD - N_SUB,), i32)
    node_cat = jnp.concatenate(
        [node_idx_gene, zpad, node_idx_dis, zpad]).astype(i32)
    feats = _node_gather(trans, node_cat)

    Wf_flat = jnp.stack([
        jnp.concatenate([gene_Wf[k] for k in range(NL)], axis=1),
        jnp.concatenate([dis_Wf[k] for k in range(NL)], axis=1),
    ])
    bf_flat = jnp.stack([gene_bf.reshape(1, HID), dis_bf.reshape(1, HID)])
    Z = _latent_proj(feats.reshape(2, NPAD, HID), Wf_flat, bf_flat)
    Zflat = Z.reshape(2 * NPAD, HID)

    npad_e = E_PAD - E

    def edge_arrays(ei, zoff):
        pad_g = jnp.full((npad_e,), zoff + PADROW, i32)
        src = jnp.concatenate([ei[0] + zoff, pad_g])
        dstz = jnp.concatenate([ei[1] + zoff, pad_g])
        return src, dstz

    sg, dzg = edge_arrays(edge_index_gene, 0)
    sd, dzd = edge_arrays(edge_index_dis, NPAD)
    src_cat = jnp.concatenate([sg, sd])
    dstz_cat = jnp.concatenate([dzg, dzd])
    targ_cat = jnp.concatenate([target_idx_gene, target_idx_dis]).astype(i32)
    zeros_init = jnp.zeros((NPAD, HID), f32)

    onum, oden = _edge_pass(Zflat, src_cat, dstz_cat, targ_cat, zeros_init)

    num_t = onum.reshape(2, T, HID)
    den_t = oden.reshape(2, T, NL)
    rep = np.zeros((NL, HID), np.float32)
    for k in range(NL):
        rep[k, k * DK:(k + 1) * DK] = 1.0
    foW = jnp.stack([gene_fcout_W, dis_fcout_W])
    fob = jnp.stack([gene_fcout_b.reshape(1, OUT_DIM),
                     dis_fcout_b.reshape(1, OUT_DIM)])
    res = _final_proj(num_t, den_t, jnp.asarray(rep), foW, fob)
    return res[0], res[1]
